# Initial kernel scaffold; baseline (speedup 1.0000x reference)
import jax
import jax.numpy as jnp
from jax.experimental import pallas as pl


def kernel(z, neighbor, W, b):
    raise NotImplementedError("write your pallas kernel here")


# speedup vs baseline: 10.1473x; 10.1473x over previous
import functools

import jax
import jax.numpy as jnp
from jax import lax
from jax.experimental import pallas as pl
from jax.experimental.pallas import tpu as pltpu
from jax.experimental.pallas import tpu_sc as plsc

_LANES = 16
_NWORK = 32


def _matvec_body(z_ref, w_ref, out_ref):
    out_ref[...] = jnp.sum(z_ref[...] * w_ref[...], axis=1, keepdims=True)


def _row_dot(z, W):
    n, d = z.shape
    blk = 1000
    return pl.pallas_call(
        _matvec_body,
        grid=(n // blk,),
        in_specs=[
            pl.BlockSpec((blk, d), lambda i: (i, 0)),
            pl.BlockSpec((1, d), lambda i: (0, 0)),
        ],
        out_specs=pl.BlockSpec((blk, 1), lambda i: (i, 0)),
        out_shape=jax.ShapeDtypeStruct((n, 1), jnp.float32),
    )(z, W)


@functools.lru_cache(maxsize=None)
def _make_sc_gather(n_rows, n_nbr, s_len):
    assert n_rows % _LANES == 0
    groups = n_rows // _LANES
    q, rem = divmod(groups, _NWORK)
    gmax = q + (1 if rem else 0)
    rows_base = q * _LANES

    mesh = plsc.VectorSubcoreMesh(core_axis_name="c", subcore_axis_name="s")

    @functools.partial(
        pl.kernel,
        out_type=jax.ShapeDtypeStruct((n_rows,), jnp.float32),
        mesh=mesh,
        compiler_params=pltpu.CompilerParams(needs_layout_passes=False),
        scratch_types=[
            pltpu.VMEM((s_len,), jnp.float32),
            pltpu.VMEM((gmax * _LANES * n_nbr,), jnp.int32),
            pltpu.VMEM((gmax * _LANES,), jnp.float32),
            pltpu.VMEM((_LANES,), jnp.float32),
        ],
    )
    def sc_gather(s_hbm, nbr_hbm, b_hbm, out_hbm, s_v, nbr_v, out_v, b_v):
        nc = mesh.num_cores
        w = lax.axis_index("s") * nc + lax.axis_index("c")
        has_extra = w < rem
        ng = jnp.where(has_extra, q + 1, q)
        base_g = q * w + jnp.minimum(w, rem)
        idx0 = base_g * _LANES * n_nbr
        nbase = rows_base * n_nbr
        row0 = base_g * _LANES

        pltpu.sync_copy(s_hbm, s_v)
        pltpu.sync_copy(b_hbm, b_v)
        pltpu.sync_copy(nbr_hbm.at[pl.ds(idx0, nbase)], nbr_v.at[pl.ds(0, nbase)])

        @pl.when(has_extra)
        def _():
            pltpu.sync_copy(nbr_hbm.at[pl.ds(idx0 + nbase, _LANES * n_nbr)],
                            nbr_v.at[pl.ds(nbase, _LANES * n_nbr)])

        lanevec = lax.iota(jnp.int32, _LANES) * n_nbr
        bvec = b_v[...]

        def group_body(k, _):
            @pl.when(k < ng)
            def _():
                kbase = k * (_LANES * n_nbr)
                acc = jnp.zeros((_LANES,), jnp.float32)
                for j in range(n_nbr):
                    nidx = plsc.load_gather(nbr_v, [lanevec + (kbase + j)])
                    acc = acc + plsc.load_gather(s_v, [nidx + 7])
                out_v[pl.ds(k * _LANES, _LANES)] = jnp.maximum(acc + bvec, 0.0)
            return 0

        lax.fori_loop(0, gmax, group_body, 0)

        pltpu.sync_copy(out_v.at[pl.ds(0, rows_base)],
                        out_hbm.at[pl.ds(row0, rows_base)])

        @pl.when(has_extra)
        def _():
            pltpu.sync_copy(out_v.at[pl.ds(rows_base, _LANES)],
                            out_hbm.at[pl.ds(row0 + rows_base, _LANES)])

    return sc_gather


def kernel(z, neighbor, W, b):
    n, d = z.shape
    s = _row_dot(z, W)
    s_full = jnp.concatenate(
        [jnp.zeros((8, 1), jnp.float32), s], axis=0).reshape(-1)
    b16 = jnp.broadcast_to(b.astype(jnp.float32), (_LANES,))
    sc = _make_sc_gather(neighbor.shape[0], neighbor.shape[1], n + 8)
    return sc(s_full, neighbor.reshape(-1), b16)


# speedup vs baseline: 11.6474x; 1.1478x over previous
import functools

import jax
import jax.numpy as jnp
from jax import lax
from jax.experimental import pallas as pl
from jax.experimental.pallas import tpu as pltpu
from jax.experimental.pallas import tpu_sc as plsc

_LANES = 16
_NWORK = 32


def _matvec_body(z_ref, w_ref, out_ref):
    out_ref[...] = jnp.sum(z_ref[...] * w_ref[...], axis=1, keepdims=True)


def _row_dot(z, W):
    n, d = z.shape
    blk = 5000
    return pl.pallas_call(
        _matvec_body,
        grid=(n // blk,),
        in_specs=[
            pl.BlockSpec((blk, d), lambda i: (i, 0)),
            pl.BlockSpec((1, d), lambda i: (0, 0)),
        ],
        out_specs=pl.BlockSpec((blk, 1), lambda i: (i, 0)),
        out_shape=jax.ShapeDtypeStruct((n, 1), jnp.float32),
    )(z, W)


@functools.lru_cache(maxsize=None)
def _make_sc_gather(n_rows, n_nbr, s_len):
    assert n_rows % _LANES == 0
    groups = n_rows // _LANES
    q, rem = divmod(groups, _NWORK)
    gmax = q + (1 if rem else 0)
    rows_base = q * _LANES

    mesh = plsc.VectorSubcoreMesh(core_axis_name="c", subcore_axis_name="s")

    @functools.partial(
        pl.kernel,
        out_type=jax.ShapeDtypeStruct((n_rows,), jnp.float32),
        mesh=mesh,
        compiler_params=pltpu.CompilerParams(needs_layout_passes=False),
        scratch_types=[
            pltpu.VMEM((s_len,), jnp.float32),
            pltpu.VMEM((gmax * _LANES * n_nbr,), jnp.int32),
            pltpu.VMEM((gmax * _LANES,), jnp.float32),
            pltpu.VMEM((_LANES,), jnp.float32),
            pltpu.SemaphoreType.DMA,
        ],
    )
    def sc_gather(s_hbm, nbr_hbm, b_hbm, out_hbm, s_v, nbr_v, out_v, b_v, sem):
        nc = mesh.num_cores
        w = lax.axis_index("s") * nc + lax.axis_index("c")
        has_extra = w < rem
        ng = jnp.where(has_extra, q + 1, q)
        base_g = q * w + jnp.minimum(w, rem)
        idx0 = base_g * _LANES * n_nbr
        nbase = rows_base * n_nbr
        row0 = base_g * _LANES

        cps = [
            pltpu.async_copy(s_hbm, s_v, sem),
            pltpu.async_copy(b_hbm, b_v, sem),
            pltpu.async_copy(nbr_hbm.at[pl.ds(idx0, nbase)],
                             nbr_v.at[pl.ds(0, nbase)], sem),
        ]

        @pl.when(has_extra)
        def _():
            pltpu.async_copy(nbr_hbm.at[pl.ds(idx0 + nbase, _LANES * n_nbr)],
                             nbr_v.at[pl.ds(nbase, _LANES * n_nbr)], sem).wait()

        for cp in cps:
            cp.wait()

        lanevec = lax.iota(jnp.int32, _LANES) * n_nbr
        bvec = b_v[...]

        def group_body(k, _):
            @pl.when(k < ng)
            def _():
                kbase = k * (_LANES * n_nbr)
                acc = jnp.zeros((_LANES,), jnp.float32)
                for j in range(n_nbr):
                    nidx = plsc.load_gather(nbr_v, [lanevec + (kbase + j)])
                    acc = acc + plsc.load_gather(s_v, [nidx + 7])
                out_v[pl.ds(k * _LANES, _LANES)] = jnp.maximum(acc + bvec, 0.0)
            return 0

        lax.fori_loop(0, gmax, group_body, 0)

        pltpu.sync_copy(out_v.at[pl.ds(0, rows_base)],
                        out_hbm.at[pl.ds(row0, rows_base)])

        @pl.when(has_extra)
        def _():
            pltpu.sync_copy(out_v.at[pl.ds(rows_base, _LANES)],
                            out_hbm.at[pl.ds(row0 + rows_base, _LANES)])

    return sc_gather


def kernel(z, neighbor, W, b):
    n, d = z.shape
    s = _row_dot(z, W)
    s_full = jnp.concatenate(
        [jnp.zeros((8, 1), jnp.float32), s], axis=0).reshape(-1)
    b16 = jnp.broadcast_to(b.astype(jnp.float32), (_LANES,))
    sc = _make_sc_gather(neighbor.shape[0], neighbor.shape[1], n + 8)
    return sc(s_full, neighbor.reshape(-1), b16)
